# Initial kernel scaffold; baseline (speedup 1.0000x reference)
import jax
import jax.numpy as jnp
from jax.experimental import pallas as pl


def kernel(x, W, b):
    raise NotImplementedError("write your pallas kernel here")


# speedup vs baseline: 1.4389x; 1.4389x over previous
import functools

import jax
import jax.numpy as jnp
from jax.experimental import pallas as pl

_N_TOK = 16384
_D_MODEL = 4096
_N_EXP = 64
_Z_COEF = 0.001
_BLOCK = 512


def _router_block(x_ref, w_ref, b_ref, probs_ref, idx_ref, z_ref, *, n_tok, block):
    logits = jnp.dot(x_ref[...], w_ref[...], preferred_element_type=jnp.float32)
    logits = logits + b_ref[...]

    iota = jax.lax.broadcasted_iota(jnp.int32, logits.shape, 1)
    m1 = jnp.max(logits, axis=1, keepdims=True)
    i1 = jnp.min(jnp.where(logits == m1, iota, _N_EXP), axis=1, keepdims=True)
    masked = jnp.where(iota == i1, -jnp.inf, logits)
    m2 = jnp.max(masked, axis=1, keepdims=True)
    i2 = jnp.min(jnp.where(masked == m2, iota, _N_EXP), axis=1, keepdims=True)

    lse = m1[:, 0] + jnp.log(jnp.sum(jnp.exp(logits - m1), axis=1))

    r = jnp.exp(m2 - m1)
    denom = 1.0 + r
    probs_ref[...] = jnp.concatenate([1.0 / denom, r / denom], axis=1)
    idx_ref[...] = jnp.concatenate([i1, i2], axis=1)

    pid = pl.program_id(0)

    @pl.when(pid == 0)
    def _init():
        z_ref[...] = jnp.zeros_like(z_ref)

    z_ref[...] += jnp.sum(lse * lse).reshape(1, 1)

    @pl.when(pid == (n_tok // block) - 1)
    def _finish():
        z_ref[...] = z_ref[...] * (_Z_COEF / n_tok)


@jax.jit
def kernel(x, W, b):
    n_tok, d_model = x.shape
    n_exp = W.shape[1]
    block = _BLOCK
    grid = (n_tok // block,)
    probs, idx, z = pl.pallas_call(
        functools.partial(_router_block, n_tok=n_tok, block=block),
        grid=grid,
        in_specs=[
            pl.BlockSpec((block, d_model), lambda i: (i, 0)),
            pl.BlockSpec((d_model, n_exp), lambda i: (0, 0)),
            pl.BlockSpec((1, n_exp), lambda i: (0, 0)),
        ],
        out_specs=[
            pl.BlockSpec((block, 2), lambda i: (i, 0)),
            pl.BlockSpec((block, 2), lambda i: (i, 0)),
            pl.BlockSpec((1, 1), lambda i: (0, 0)),
        ],
        out_shape=[
            jax.ShapeDtypeStruct((n_tok, 2), jnp.float32),
            jax.ShapeDtypeStruct((n_tok, 2), jnp.int32),
            jax.ShapeDtypeStruct((1, 1), jnp.float32),
        ],
    )(x, W.astype(jnp.float32), b.reshape(1, n_exp).astype(jnp.float32))
    return probs, idx, z[0, 0]


# speedup vs baseline: 1.5558x; 1.0812x over previous
import functools

import jax
import jax.numpy as jnp
from jax.experimental import pallas as pl

_N_TOK = 16384
_D_MODEL = 4096
_N_EXP = 64
_Z_COEF = 0.001
_BLOCK = 1024


def _router_block(x_ref, w_ref, b_ref, probs_ref, idx_ref, z_ref, *, n_tok, block):
    logits = jnp.dot(x_ref[...], w_ref[...], preferred_element_type=jnp.float32)
    logits = logits + b_ref[...]

    iota = jax.lax.broadcasted_iota(jnp.int32, logits.shape, 1)
    m1 = jnp.max(logits, axis=1, keepdims=True)
    i1 = jnp.min(jnp.where(logits == m1, iota, _N_EXP), axis=1, keepdims=True)
    masked = jnp.where(iota == i1, -jnp.inf, logits)
    m2 = jnp.max(masked, axis=1, keepdims=True)
    i2 = jnp.min(jnp.where(masked == m2, iota, _N_EXP), axis=1, keepdims=True)

    lse = m1[:, 0] + jnp.log(jnp.sum(jnp.exp(logits - m1), axis=1))

    r = jnp.exp(m2 - m1)
    denom = 1.0 + r
    probs_ref[...] = jnp.concatenate([1.0 / denom, r / denom], axis=1)
    idx_ref[...] = jnp.concatenate([i1, i2], axis=1)

    pid = pl.program_id(0)

    @pl.when(pid == 0)
    def _init():
        z_ref[...] = jnp.zeros_like(z_ref)

    z_ref[...] += jnp.sum(lse * lse).reshape(1, 1)

    @pl.when(pid == (n_tok // block) - 1)
    def _finish():
        z_ref[...] = z_ref[...] * (_Z_COEF / n_tok)


@jax.jit
def kernel(x, W, b):
    n_tok, d_model = x.shape
    n_exp = W.shape[1]
    block = _BLOCK
    grid = (n_tok // block,)
    probs, idx, z = pl.pallas_call(
        functools.partial(_router_block, n_tok=n_tok, block=block),
        grid=grid,
        in_specs=[
            pl.BlockSpec((block, d_model), lambda i: (i, 0)),
            pl.BlockSpec((d_model, n_exp), lambda i: (0, 0)),
            pl.BlockSpec((1, n_exp), lambda i: (0, 0)),
        ],
        out_specs=[
            pl.BlockSpec((block, 2), lambda i: (i, 0)),
            pl.BlockSpec((block, 2), lambda i: (i, 0)),
            pl.BlockSpec((1, 1), lambda i: (0, 0)),
        ],
        out_shape=[
            jax.ShapeDtypeStruct((n_tok, 2), jnp.float32),
            jax.ShapeDtypeStruct((n_tok, 2), jnp.int32),
            jax.ShapeDtypeStruct((1, 1), jnp.float32),
        ],
    )(x, W.astype(jnp.float32), b.reshape(1, n_exp).astype(jnp.float32))
    return probs, idx, z[0, 0]
